# Initial kernel scaffold; baseline (speedup 1.0000x reference)
import jax
import jax.numpy as jnp
from jax.experimental import pallas as pl


def kernel(X, edge_index, edge_weight, Wu, bu, Wr, br, Wc, bc, Wlu, blu, Wlr, blr, Wlc, blc):
    raise NotImplementedError("write your pallas kernel here")


# speedup vs baseline: 12.6129x; 12.6129x over previous
import functools

import jax
import jax.numpy as jnp
from jax import lax
from jax.experimental import pallas as pl
from jax.experimental.pallas import tpu as pltpu
from jax.experimental.pallas import tpu_sc as plsc

_F32 = jnp.float32
_BLK_E = 128
_N_SUB = 16
_N_CORES = 2


def _tc_fold(Wu, Wlu, bu, blu, Wc, Wlc, bc, blc):
    d_in = Wu.shape[0]
    d_out = Wlu.shape[1]

    def body(wu, wlu, bu2, blu2, wc, wlc, bc2, blc2, wf, bf):
        wlu_t = wlu[...][:d_out, :]
        wlc_t = wlc[...][:d_out, :]
        wf[:, :d_out] = jnp.dot(wu[...], wlu_t, preferred_element_type=_F32)
        wf[:, d_out:] = jnp.dot(wc[...], wlc_t, preferred_element_type=_F32)
        bf[0:1, :] = jnp.dot(bu2[...], wlu_t, preferred_element_type=_F32) + blu2[...]
        bf[1:2, :] = jnp.dot(bc2[...], wlc_t, preferred_element_type=_F32) + blc2[...]

    return pl.pallas_call(
        body,
        out_shape=(jax.ShapeDtypeStruct((d_in, 2 * d_out), _F32),
                   jax.ShapeDtypeStruct((2, d_out), _F32)),
    )(Wu, Wlu, bu.reshape(1, -1), blu.reshape(1, -1),
      Wc, Wlc, bc.reshape(1, -1), blc.reshape(1, -1))


def _tc_dense(X, Wf, blk):
    n, d_in = X.shape
    d2 = Wf.shape[1]

    def body(x, wf, y):
        y[...] = jnp.dot(x[...], wf[...], preferred_element_type=_F32)

    return pl.pallas_call(
        body,
        grid=(n // blk,),
        in_specs=[pl.BlockSpec((blk, d_in), lambda i: (i, 0)),
                  pl.BlockSpec((d_in, d2), lambda i: (0, 0))],
        out_specs=pl.BlockSpec((blk, d2), lambda i: (i, 0)),
        out_shape=jax.ShapeDtypeStruct((n, d2), _F32),
    )(X, Wf)


def _tc_scale(degp3, Y, blk):
    n, d2 = Y.shape
    d = d2 // 2

    def body(dp, y, z, dv):
        deg = dp[0, :, 0] + dp[1, :, 0] + 1.0
        dinv = jnp.where(deg > 0, lax.rsqrt(jnp.maximum(deg, 1e-12)), 0.0)
        dv[:, 0] = dinv
        yv = y[...]
        z[0] = yv[:, :d] * dinv[:, None]
        z[1] = yv[:, d:] * dinv[:, None]

    return pl.pallas_call(
        body,
        grid=(n // blk,),
        in_specs=[pl.BlockSpec((2, blk, 1), lambda i: (0, i, 0)),
                  pl.BlockSpec((blk, d2), lambda i: (i, 0))],
        out_specs=(pl.BlockSpec((2, blk, d), lambda i: (0, i, 0)),
                   pl.BlockSpec((blk, 1), lambda i: (i, 0))),
        out_shape=(jax.ShapeDtypeStruct((2, n, d), _F32),
                   jax.ShapeDtypeStruct((n, 1), _F32)),
    )(degp3, Y)


def _tc_final(S, Z, dinv, bf, blk):
    _, n, d = S.shape

    def body(sref, zref, dv, bfr, out):
        di = dv[:, 0][:, None]
        u = jax.nn.sigmoid(di * (sref[0] + zref[0]) + bfr[0:1, :])
        c = jnp.tanh(di * (sref[1] + zref[1]) + bfr[1:2, :])
        out[...] = (1.0 - u) * c

    return pl.pallas_call(
        body,
        grid=(n // blk,),
        in_specs=[pl.BlockSpec((2, blk, d), lambda i: (0, i, 0)),
                  pl.BlockSpec((2, blk, d), lambda i: (0, i, 0)),
                  pl.BlockSpec((blk, 1), lambda i: (i, 0)),
                  pl.BlockSpec((2, d), lambda i: (0, 0))],
        out_specs=pl.BlockSpec((blk, d), lambda i: (i, 0)),
        out_shape=jax.ShapeDtypeStruct((n, d), _F32),
    )(S, Z, dinv, bf)


def _sc_deg(dst_pad, w_pad, n_pad):
    e_pad = dst_pad.shape[0]
    nblk = e_pad // (_N_CORES * _N_SUB * _BLK_E)
    stripe = n_pad // _N_SUB
    mesh = plsc.VectorSubcoreMesh(core_axis_name="c", subcore_axis_name="s")

    @functools.partial(
        pl.kernel, mesh=mesh,
        out_type=jax.ShapeDtypeStruct((_N_CORES, n_pad), _F32),
        scratch_types=[
            pltpu.VMEM((stripe,), _F32),
            pltpu.VMEM((_BLK_E,), jnp.int32),
            pltpu.VMEM((_BLK_E,), _F32),
            pltpu.VMEM_SHARED((n_pad,), _F32),
        ],
    )
    def k(dst_hbm, w_hbm, out_hbm, zbuf, dvec, wvec, acc):
        c = lax.axis_index("c")
        s = lax.axis_index("s")

        @pl.loop(0, stripe, step=16)
        def _(i):
            zbuf[pl.ds(i, 16)] = jnp.zeros((16,), _F32)

        pltpu.sync_copy(zbuf, acc.at[pl.ds(s * stripe, stripe)])
        plsc.subcore_barrier()

        wid = c * _N_SUB + s

        @pl.loop(0, nblk)
        def _(b):
            off = (wid * nblk + b) * _BLK_E
            pltpu.sync_copy(dst_hbm.at[pl.ds(off, _BLK_E)], dvec)
            pltpu.sync_copy(w_hbm.at[pl.ds(off, _BLK_E)], wvec)
            pltpu.sync_copy(wvec, acc.at[dvec], add=True)

        plsc.subcore_barrier()
        pltpu.sync_copy(acc.at[pl.ds(s * stripe, stripe)],
                        out_hbm.at[c, pl.ds(s * stripe, stripe)])

    return k(dst_pad, w_pad)


def _sc_spmm(z_stack, src_pad, dst_pad, w_pad, n_acc):
    _, n, d = z_stack.shape
    e_pad = src_pad.shape[0]
    nblk = e_pad // (_N_SUB * _BLK_E)
    stripe = n_acc // _N_SUB
    zrows = 128
    mesh = plsc.VectorSubcoreMesh(core_axis_name="c", subcore_axis_name="s")

    @functools.partial(
        pl.kernel, mesh=mesh,
        out_type=jax.ShapeDtypeStruct((_N_CORES, n_acc, d), _F32),
        scratch_types=[
            pltpu.VMEM((zrows, d), _F32),
            pltpu.VMEM((_BLK_E,), jnp.int32),
            pltpu.VMEM((_BLK_E,), jnp.int32),
            pltpu.VMEM((_BLK_E,), _F32),
            pltpu.VMEM((_BLK_E, d), _F32),
            pltpu.VMEM_SHARED((n_acc, d), _F32),
        ],
    )
    def k(z_hbm, src_hbm, dst_hbm, w_hbm, out_hbm,
          ztile, svec, dvec, wvec, rows, acc):
        c = lax.axis_index("c")
        s = lax.axis_index("s")

        @pl.loop(0, zrows)
        def _(r):
            for kk in range(d // 16):
                ztile[r, pl.ds(kk * 16, 16)] = jnp.zeros((16,), _F32)

        for rep in range(stripe // zrows):
            pltpu.sync_copy(ztile, acc.at[pl.ds(s * stripe + rep * zrows, zrows), :])
        plsc.subcore_barrier()

        @pl.loop(0, nblk)
        def _(b):
            off = (s * nblk + b) * _BLK_E
            pltpu.sync_copy(src_hbm.at[pl.ds(off, _BLK_E)], svec)
            pltpu.sync_copy(dst_hbm.at[pl.ds(off, _BLK_E)], dvec)
            pltpu.sync_copy(w_hbm.at[pl.ds(off, _BLK_E)], wvec)
            pltpu.sync_copy(z_hbm.at[c].at[svec], rows)

            @pl.loop(0, _BLK_E, step=16)
            def _(r0):
                wv = wvec[pl.ds(r0, 16)]
                for j in range(16):
                    wr = wv[j]
                    for kk in range(d // 16):
                        sl = pl.ds(kk * 16, 16)
                        rows[r0 + j, sl] = rows[r0 + j, sl] * wr

            pltpu.sync_copy(rows, acc.at[dvec], add=True)

        plsc.subcore_barrier()
        pltpu.sync_copy(acc.at[pl.ds(s * stripe, stripe), :],
                        out_hbm.at[c, pl.ds(s * stripe, stripe), :])

    return k(z_stack, src_pad, dst_pad, w_pad)


def kernel(X, edge_index, edge_weight, Wu, bu, Wr, br, Wc, bc,
           Wlu, blu, Wlr, blr, Wlc, blc):
    n = X.shape[0]
    e = edge_weight.shape[0]
    src = edge_index[0]
    dst = edge_index[1]

    grain = _N_CORES * _N_SUB * _BLK_E
    e_pad = ((e + grain - 1) // grain) * grain
    pad = e_pad - e
    src_p = jnp.pad(src, (0, pad))
    dst_p = jnp.pad(dst, (0, pad))
    w_p = jnp.pad(edge_weight, (0, pad))

    sub_grain = _N_SUB * 16
    n_pad = ((n + sub_grain - 1) // sub_grain) * sub_grain

    acc_grain = _N_SUB * 128
    n_acc = ((n + acc_grain - 1) // acc_grain) * acc_grain

    wf, bf = _tc_fold(Wu, Wlu, bu, blu, Wc, Wlc, bc, blc)
    y = _tc_dense(X, wf, blk=1000)
    degp = _sc_deg(dst_p, w_p, n_pad)
    degp3 = degp[:, :n, None]
    z, dinv = _tc_scale(degp3, y, blk=1000)
    s_acc = _sc_spmm(z, src_p, dst_p, w_p, n_acc)
    return _tc_final(s_acc[:, :n, :], z, dinv, bf, blk=1000)


# speedup vs baseline: 22.6165x; 1.7931x over previous
import functools

import jax
import jax.numpy as jnp
from jax import lax
from jax.experimental import pallas as pl
from jax.experimental.pallas import tpu as pltpu
from jax.experimental.pallas import tpu_sc as plsc

_F32 = jnp.float32
_BLK_E = 128
_N_SUB = 16
_N_CORES = 2


def _tc_fold(Wu, Wlu, bu, blu, Wc, Wlc, bc, blc):
    d_in = Wu.shape[0]
    d_out = Wlu.shape[1]

    def body(wu, wlu, bu2, blu2, wc, wlc, bc2, blc2, wf, bf):
        wlu_t = wlu[...][:d_out, :]
        wlc_t = wlc[...][:d_out, :]
        wf[:, :d_out] = jnp.dot(wu[...], wlu_t, preferred_element_type=_F32)
        wf[:, d_out:] = jnp.dot(wc[...], wlc_t, preferred_element_type=_F32)
        bf[0:1, :] = jnp.dot(bu2[...], wlu_t, preferred_element_type=_F32) + blu2[...]
        bf[1:2, :] = jnp.dot(bc2[...], wlc_t, preferred_element_type=_F32) + blc2[...]

    return pl.pallas_call(
        body,
        out_shape=(jax.ShapeDtypeStruct((d_in, 2 * d_out), _F32),
                   jax.ShapeDtypeStruct((2, d_out), _F32)),
    )(Wu, Wlu, bu.reshape(1, -1), blu.reshape(1, -1),
      Wc, Wlc, bc.reshape(1, -1), blc.reshape(1, -1))


def _tc_xd(degp3, X, blk):
    n, d = X.shape

    def body(dp, x, xd, dv):
        deg = dp[0, :, 0] + dp[1, :, 0] + 1.0
        dinv = jnp.where(deg > 0, lax.rsqrt(jnp.maximum(deg, 1e-12)), 0.0)
        dv[:, 0] = dinv
        xd[...] = x[...] * dinv[:, None]

    return pl.pallas_call(
        body,
        grid=(n // blk,),
        in_specs=[pl.BlockSpec((2, blk, 1), lambda i: (0, i, 0)),
                  pl.BlockSpec((blk, d), lambda i: (i, 0))],
        out_specs=(pl.BlockSpec((blk, d), lambda i: (i, 0)),
                   pl.BlockSpec((blk, 1), lambda i: (i, 0))),
        out_shape=(jax.ShapeDtypeStruct((n, d), _F32),
                   jax.ShapeDtypeStruct((n, 1), _F32)),
    )(degp3, X)


def _tc_final(P2, Xd, dinv, wf, bf, blk):
    _, n, d = P2.shape

    def body(pref, xdref, dv, wfr, bfr, out):
        t = pref[0] + pref[1] + xdref[...]
        m = jnp.dot(t, wfr[...], preferred_element_type=_F32)
        di = dv[:, 0][:, None]
        u = jax.nn.sigmoid(di * m[:, :d] + bfr[0:1, :])
        c = jnp.tanh(di * m[:, d:] + bfr[1:2, :])
        out[...] = (1.0 - u) * c

    return pl.pallas_call(
        body,
        grid=(n // blk,),
        in_specs=[pl.BlockSpec((2, blk, d), lambda i: (0, i, 0)),
                  pl.BlockSpec((blk, d), lambda i: (i, 0)),
                  pl.BlockSpec((blk, 1), lambda i: (i, 0)),
                  pl.BlockSpec((d, 2 * d), lambda i: (0, 0)),
                  pl.BlockSpec((2, d), lambda i: (0, 0))],
        out_specs=pl.BlockSpec((blk, d), lambda i: (i, 0)),
        out_shape=jax.ShapeDtypeStruct((n, d), _F32),
    )(P2, Xd, dinv, wf, bf)


def _sc_deg(dst2, w2, n_pad):
    rows_total = dst2.shape[0]
    nblk = rows_total // (_N_CORES * _N_SUB)
    stripe = n_pad // _N_SUB
    mesh = plsc.VectorSubcoreMesh(core_axis_name="c", subcore_axis_name="s")

    @functools.partial(
        pl.kernel, mesh=mesh,
        out_type=jax.ShapeDtypeStruct((_N_CORES, n_pad), _F32),
        scratch_types=[
            pltpu.VMEM((stripe,), _F32),
            pltpu.VMEM((nblk, _BLK_E), jnp.int32),
            pltpu.VMEM((nblk, _BLK_E), _F32),
            pltpu.VMEM_SHARED((n_pad,), _F32),
            pltpu.SemaphoreType.DMA,
            pltpu.SemaphoreType.DMA,
            pltpu.SemaphoreType.DMA,
        ],
    )
    def k(dst_hbm, w_hbm, out_hbm, zbuf, dmat, wmat, acc, sem_d, sem_w, sem_s):
        c = lax.axis_index("c")
        s = lax.axis_index("s")
        wid = c * _N_SUB + s

        pltpu.async_copy(dst_hbm.at[pl.ds(wid * nblk, nblk), :], dmat, sem_d)
        pltpu.async_copy(w_hbm.at[pl.ds(wid * nblk, nblk), :], wmat, sem_w)

        @pl.loop(0, stripe, step=16)
        def _(i):
            zbuf[pl.ds(i, 16)] = jnp.zeros((16,), _F32)

        pltpu.sync_copy(zbuf, acc.at[pl.ds(s * stripe, stripe)])
        pltpu.make_async_copy(dst_hbm.at[pl.ds(wid * nblk, nblk), :], dmat,
                              sem_d).wait()
        pltpu.make_async_copy(w_hbm.at[pl.ds(wid * nblk, nblk), :], wmat,
                              sem_w).wait()
        plsc.subcore_barrier()

        fire = 4

        @pl.loop(0, nblk, step=fire)
        def _(b):
            for i in range(fire):
                pltpu.async_copy(wmat.at[b + i], acc.at[dmat.at[b + i]],
                                 sem_s, add=True)
            for i in range(fire):
                pltpu.make_async_copy(wmat.at[b + i], acc.at[dmat.at[b + i]],
                                      sem_s).wait()

        plsc.subcore_barrier()
        pltpu.sync_copy(acc.at[pl.ds(s * stripe, stripe)],
                        out_hbm.at[c, pl.ds(s * stripe, stripe)])

    return k(dst2, w2)


def _sc_spmm(xd, src64, dst64, w64, n_acc):
    n, d = xd.shape
    blk = 64
    ch_blk = 8
    nbuf = 4
    rows_total = src64.shape[0]
    nblk = rows_total // (_N_CORES * _N_SUB)
    nchunk = nblk // ch_blk
    stripe = n_acc // _N_SUB
    mesh = plsc.VectorSubcoreMesh(core_axis_name="c", subcore_axis_name="s")

    @functools.partial(
        pl.kernel, mesh=mesh,
        out_type=jax.ShapeDtypeStruct((_N_CORES, n_acc, d), _F32),
        scratch_types=[
            pltpu.VMEM((2, ch_blk, blk), jnp.int32),
            pltpu.VMEM((2, ch_blk, blk), jnp.int32),
            pltpu.VMEM((2, ch_blk, blk), _F32),
            pltpu.VMEM((nbuf, blk, d), _F32),
            pltpu.VMEM_SHARED((n_acc, d), _F32),
        ] + [pltpu.SemaphoreType.DMA] * (2 + 2 * nbuf),
    )
    def k(xd_hbm, src_hbm, dst_hbm, w_hbm, out_hbm,
          smat, dmat, wmat, rows, acc, *sems):
        isem = sems[:2]
        gsem = sems[2:2 + nbuf]
        ssem = sems[2 + nbuf:]
        c = lax.axis_index("c")
        s = lax.axis_index("s")
        wid = c * _N_SUB + s

        def idx_load(ch, p):
            off = wid * nblk + ch * ch_blk
            pltpu.async_copy(src_hbm.at[pl.ds(off, ch_blk), :], smat.at[p], isem[0])
            pltpu.async_copy(dst_hbm.at[pl.ds(off, ch_blk), :], dmat.at[p], isem[1])
            pltpu.async_copy(w_hbm.at[pl.ds(off, ch_blk), :], wmat.at[p], isem[0])

        def idx_wait(ch, p):
            off = wid * nblk + ch * ch_blk
            pltpu.make_async_copy(src_hbm.at[pl.ds(off, ch_blk), :],
                                  smat.at[p], isem[0]).wait()
            pltpu.make_async_copy(dst_hbm.at[pl.ds(off, ch_blk), :],
                                  dmat.at[p], isem[1]).wait()
            pltpu.make_async_copy(w_hbm.at[pl.ds(off, ch_blk), :],
                                  wmat.at[p], isem[0]).wait()

        def gather(p, t, q):
            pltpu.async_copy(xd_hbm.at[smat.at[p, t]], rows.at[q], gsem[q])

        def gather_wait(p, t, q):
            pltpu.make_async_copy(xd_hbm.at[smat.at[p, t]], rows.at[q],
                                  gsem[q]).wait()

        def scatter(p, t, q):
            pltpu.async_copy(rows.at[q], acc.at[dmat.at[p, t]], ssem[q],
                             add=True)

        def scatter_wait(p, t, q):
            pltpu.make_async_copy(rows.at[q], acc.at[dmat.at[p, t]],
                                  ssem[q]).wait()

        idx_load(0, 0)

        @pl.loop(0, blk)
        def _(r):
            for kk in range(d // 16):
                rows[0, r, pl.ds(kk * 16, 16)] = jnp.zeros((16,), _F32)

        for rep in range(stripe // blk):
            pltpu.sync_copy(rows.at[0],
                            acc.at[pl.ds(s * stripe + rep * blk, blk), :])

        idx_wait(0, 0)
        gather(0, 0, 0)
        gather(0, 1, 1)
        plsc.subcore_barrier()

        @pl.loop(0, nchunk)
        def _(ch):
            p = lax.rem(ch, 2)
            pn = 1 - p
            for t in range(ch_blk):
                j = ch * ch_blk + t
                q = t % nbuf
                qn = (t + 2) % nbuf
                gather_wait(p, t, q)

                @pl.loop(0, blk, step=16)
                def _(r0):
                    wv = wmat[p, t, pl.ds(r0, 16)]
                    for l in range(16):
                        wr = wv[l]
                        for kk in range(d // 16):
                            sl = pl.ds(kk * 16, 16)
                            rows[q, r0 + l, sl] = rows[q, r0 + l, sl] * wr

                scatter(p, t, q)

                @pl.when(j >= 2)
                def _():
                    scatter_wait(p, t, qn)

                if t == 2:
                    @pl.when(ch + 1 < nchunk)
                    def _():
                        idx_load(ch + 1, pn)

                if t < ch_blk - 2:
                    @pl.when(j + 2 < nblk)
                    def _():
                        gather(p, t + 2, qn)
                else:
                    if t == ch_blk - 2:
                        @pl.when(ch + 1 < nchunk)
                        def _():
                            idx_wait(ch + 1, pn)

                    @pl.when(ch + 1 < nchunk)
                    def _():
                        gather(pn, t - (ch_blk - 2), qn)

        scatter_wait((nchunk - 1) % 2, ch_blk - 2, (nblk - 2) % nbuf)
        scatter_wait((nchunk - 1) % 2, ch_blk - 1, (nblk - 1) % nbuf)
        plsc.subcore_barrier()
        pltpu.sync_copy(acc.at[pl.ds(s * stripe, stripe), :],
                        out_hbm.at[c, pl.ds(s * stripe, stripe), :])

    return k(xd, src64, dst64, w64)


def kernel(X, edge_index, edge_weight, Wu, bu, Wr, br, Wc, bc,
           Wlu, blu, Wlr, blr, Wlc, blc):
    n = X.shape[0]
    e = edge_weight.shape[0]
    src = edge_index[0]
    dst = edge_index[1]

    grain = 256 * _BLK_E
    e_pad = ((e + grain - 1) // grain) * grain
    pad = e_pad - e
    src_p = jnp.pad(src, (0, pad))
    dst_p = jnp.pad(dst, (0, pad))
    w_p = jnp.pad(edge_weight, (0, pad))
    dst2 = dst_p.reshape(-1, _BLK_E)
    w2 = w_p.reshape(-1, _BLK_E)
    src64 = src_p.reshape(-1, 64)
    dst64 = dst_p.reshape(-1, 64)
    w64 = w_p.reshape(-1, 64)

    sub_grain = _N_SUB * 16
    n_pad = ((n + sub_grain - 1) // sub_grain) * sub_grain

    acc_grain = _N_SUB * 128
    n_acc = ((n + acc_grain - 1) // acc_grain) * acc_grain

    wf, bf = _tc_fold(Wu, Wlu, bu, blu, Wc, Wlc, bc, blc)
    degp = _sc_deg(dst2, w2, n_pad)
    degp3 = degp[:, :n, None]
    xd, dinv = _tc_xd(degp3, X, blk=1000)
    p2 = _sc_spmm(xd, src64, dst64, w64, n_acc)
    return _tc_final(p2[:, :n, :], xd, dinv, wf, bf, blk=1000)


# speedup vs baseline: 53.8533x; 2.3812x over previous
import functools

import jax
import jax.numpy as jnp
from jax import lax
from jax.experimental import pallas as pl
from jax.experimental.pallas import tpu as pltpu
from jax.experimental.pallas import tpu_sc as plsc

_F32 = jnp.float32
_BLK_E = 128
_N_SUB = 16
_N_CORES = 2


def _tc_fold(Wu, Wlu, bu, blu, Wc, Wlc, bc, blc):
    d_in = Wu.shape[0]
    d_out = Wlu.shape[1]

    def body(wu, wlu, bu2, blu2, wc, wlc, bc2, blc2, wf, bf):
        wlu_t = wlu[...][:d_out, :]
        wlc_t = wlc[...][:d_out, :]
        wf[:, :d_out] = jnp.dot(wu[...], wlu_t, preferred_element_type=_F32)
        wf[:, d_out:] = jnp.dot(wc[...], wlc_t, preferred_element_type=_F32)
        bf[0:1, :] = jnp.dot(bu2[...], wlu_t, preferred_element_type=_F32) + blu2[...]
        bf[1:2, :] = jnp.dot(bc2[...], wlc_t, preferred_element_type=_F32) + blc2[...]

    return pl.pallas_call(
        body,
        out_shape=(jax.ShapeDtypeStruct((d_in, 2 * d_out), _F32),
                   jax.ShapeDtypeStruct((2, d_out), _F32)),
    )(Wu, Wlu, bu.reshape(1, -1), blu.reshape(1, -1),
      Wc, Wlc, bc.reshape(1, -1), blc.reshape(1, -1))


def _tc_xd(degp3, X, blk):
    n, d = X.shape

    def body(dp, x, xd, dv):
        deg = dp[0, :, 0] + dp[1, :, 0] + 1.0
        dinv = jnp.where(deg > 0, lax.rsqrt(jnp.maximum(deg, 1e-12)), 0.0)
        dv[:, 0] = dinv
        xd[...] = x[...] * dinv[:, None]

    return pl.pallas_call(
        body,
        grid=(n // blk,),
        in_specs=[pl.BlockSpec((2, blk, 1), lambda i: (0, i, 0)),
                  pl.BlockSpec((blk, d), lambda i: (i, 0))],
        out_specs=(pl.BlockSpec((blk, d), lambda i: (i, 0)),
                   pl.BlockSpec((blk, 1), lambda i: (i, 0))),
        out_shape=(jax.ShapeDtypeStruct((n, d), _F32),
                   jax.ShapeDtypeStruct((n, 1), _F32)),
    )(degp3, X)


def _tc_final(P2, Xd, dinv, wf, bf, blk):
    _, n, d = P2.shape

    def body(pref, xdref, dv, wfr, bfr, out):
        t = pref[0] + pref[1] + xdref[...]
        m = jnp.dot(t, wfr[...], preferred_element_type=_F32)
        di = dv[:, 0][:, None]
        u = jax.nn.sigmoid(di * m[:, :d] + bfr[0:1, :])
        c = jnp.tanh(di * m[:, d:] + bfr[1:2, :])
        out[...] = (1.0 - u) * c

    return pl.pallas_call(
        body,
        grid=(n // blk,),
        in_specs=[pl.BlockSpec((2, blk, d), lambda i: (0, i, 0)),
                  pl.BlockSpec((blk, d), lambda i: (i, 0)),
                  pl.BlockSpec((blk, 1), lambda i: (i, 0)),
                  pl.BlockSpec((d, 2 * d), lambda i: (0, 0)),
                  pl.BlockSpec((2, d), lambda i: (0, 0))],
        out_specs=pl.BlockSpec((blk, d), lambda i: (i, 0)),
        out_shape=jax.ShapeDtypeStruct((n, d), _F32),
    )(P2, Xd, dinv, wf, bf)


def _sc_deg(dst2, w2, n_pad):
    rows_total = dst2.shape[0]
    nblk = rows_total // (_N_CORES * _N_SUB)
    stripe = n_pad // _N_SUB
    mesh = plsc.VectorSubcoreMesh(core_axis_name="c", subcore_axis_name="s")

    @functools.partial(
        pl.kernel, mesh=mesh,
        out_type=jax.ShapeDtypeStruct((_N_CORES, n_pad), _F32),
        scratch_types=[
            pltpu.VMEM((stripe,), _F32),
            pltpu.VMEM((nblk, _BLK_E), jnp.int32),
            pltpu.VMEM((nblk, _BLK_E), _F32),
            pltpu.VMEM_SHARED((n_pad,), _F32),
            pltpu.SemaphoreType.DMA,
            pltpu.SemaphoreType.DMA,
            pltpu.SemaphoreType.DMA,
        ],
    )
    def k(dst_hbm, w_hbm, out_hbm, zbuf, dmat, wmat, acc, sem_d, sem_w, sem_s):
        c = lax.axis_index("c")
        s = lax.axis_index("s")
        wid = c * _N_SUB + s

        pltpu.async_copy(dst_hbm.at[pl.ds(wid * nblk, nblk), :], dmat, sem_d)
        pltpu.async_copy(w_hbm.at[pl.ds(wid * nblk, nblk), :], wmat, sem_w)

        @pl.loop(0, stripe, step=16)
        def _(i):
            zbuf[pl.ds(i, 16)] = jnp.zeros((16,), _F32)

        pltpu.sync_copy(zbuf, acc.at[pl.ds(s * stripe, stripe)])
        pltpu.make_async_copy(dst_hbm.at[pl.ds(wid * nblk, nblk), :], dmat,
                              sem_d).wait()
        pltpu.make_async_copy(w_hbm.at[pl.ds(wid * nblk, nblk), :], wmat,
                              sem_w).wait()
        plsc.subcore_barrier()

        fire = 4

        @pl.loop(0, nblk, step=fire)
        def _(b):
            for i in range(fire):
                pltpu.async_copy(wmat.at[b + i], acc.at[dmat.at[b + i]],
                                 sem_s, add=True)
            for i in range(fire):
                pltpu.make_async_copy(wmat.at[b + i], acc.at[dmat.at[b + i]],
                                      sem_s).wait()

        plsc.subcore_barrier()
        pltpu.sync_copy(acc.at[pl.ds(s * stripe, stripe)],
                        out_hbm.at[c, pl.ds(s * stripe, stripe)])

    return k(dst2, w2)


def _sc_spmm(xd, src64, dst64, w64, n_acc):
    n, d = xd.shape
    blk = 64
    ch_blk = 8
    nbuf = 4
    rows_total = src64.shape[0]
    nblk = rows_total // (_N_CORES * _N_SUB)
    nchunk = nblk // ch_blk
    stripe = n_acc // _N_SUB
    mesh = plsc.VectorSubcoreMesh(core_axis_name="c", subcore_axis_name="s")

    @functools.partial(
        pl.kernel, mesh=mesh,
        out_type=jax.ShapeDtypeStruct((_N_CORES, n_acc, d), _F32),
        scratch_types=[
            pltpu.VMEM((2, ch_blk, blk), jnp.int32),
            pltpu.VMEM((2, ch_blk, blk), jnp.int32),
            pltpu.VMEM((2, ch_blk, blk), _F32),
            pltpu.VMEM((nbuf, blk, d), _F32),
            pltpu.VMEM_SHARED((n_acc, d), _F32),
        ] + [pltpu.SemaphoreType.DMA] * (2 + 2 * nbuf),
    )
    def k(xd_hbm, src_hbm, dst_hbm, w_hbm, out_hbm,
          smat, dmat, wmat, rows, acc, *sems):
        isem = sems[:2]
        gsem = sems[2:2 + nbuf]
        ssem = sems[2 + nbuf:]
        c = lax.axis_index("c")
        s = lax.axis_index("s")
        wid = c * _N_SUB + s

        def idx_load(ch, p):
            off = wid * nblk + ch * ch_blk
            pltpu.async_copy(src_hbm.at[pl.ds(off, ch_blk), :], smat.at[p], isem[0])
            pltpu.async_copy(dst_hbm.at[pl.ds(off, ch_blk), :], dmat.at[p], isem[1])
            pltpu.async_copy(w_hbm.at[pl.ds(off, ch_blk), :], wmat.at[p], isem[0])

        def idx_wait(ch, p):
            off = wid * nblk + ch * ch_blk
            pltpu.make_async_copy(src_hbm.at[pl.ds(off, ch_blk), :],
                                  smat.at[p], isem[0]).wait()
            pltpu.make_async_copy(dst_hbm.at[pl.ds(off, ch_blk), :],
                                  dmat.at[p], isem[1]).wait()
            pltpu.make_async_copy(w_hbm.at[pl.ds(off, ch_blk), :],
                                  wmat.at[p], isem[0]).wait()

        def gather(p, t, q):
            pltpu.async_copy(xd_hbm.at[smat.at[p, t]], rows.at[q], gsem[q])

        def gather_wait(p, t, q):
            pltpu.make_async_copy(xd_hbm.at[smat.at[p, t]], rows.at[q],
                                  gsem[q]).wait()

        def scatter(p, t, q):
            pltpu.async_copy(rows.at[q], acc.at[dmat.at[p, t]], ssem[q],
                             add=True)

        def scatter_wait(p, t, q):
            pltpu.make_async_copy(rows.at[q], acc.at[dmat.at[p, t]],
                                  ssem[q]).wait()

        idx_load(0, 0)

        @pl.loop(0, blk)
        def _(r):
            for kk in range(d // 16):
                rows[0, r, pl.ds(kk * 16, 16)] = jnp.zeros((16,), _F32)

        for rep in range(stripe // blk):
            pltpu.sync_copy(rows.at[0],
                            acc.at[pl.ds(s * stripe + rep * blk, blk), :])

        idx_wait(0, 0)
        gather(0, 0, 0)
        gather(0, 1, 1)
        plsc.subcore_barrier()

        @pl.loop(0, nchunk)
        def _(ch):
            p = lax.rem(ch, 2)
            pn = 1 - p
            for t in range(ch_blk):
                j = ch * ch_blk + t
                q = t % nbuf
                qn = (t + 2) % nbuf
                gather_wait(p, t, q)

                @pl.loop(0, blk, step=16)
                def _(r0):
                    wv = wmat[p, t, pl.ds(r0, 16)]
                    for l in range(16):
                        wr = wv[l]
                        for kk in range(d // 16):
                            sl = pl.ds(kk * 16, 16)
                            rows[q, r0 + l, sl] = rows[q, r0 + l, sl] * wr

                scatter(p, t, q)

                @pl.when(j >= 2)
                def _():
                    scatter_wait(p, t, qn)

                if t == 2:
                    @pl.when(ch + 1 < nchunk)
                    def _():
                        idx_load(ch + 1, pn)

                if t < ch_blk - 2:
                    @pl.when(j + 2 < nblk)
                    def _():
                        gather(p, t + 2, qn)
                else:
                    if t == ch_blk - 2:
                        @pl.when(ch + 1 < nchunk)
                        def _():
                            idx_wait(ch + 1, pn)

                    @pl.when(ch + 1 < nchunk)
                    def _():
                        gather(pn, t - (ch_blk - 2), qn)

        scatter_wait((nchunk - 1) % 2, ch_blk - 2, (nblk - 2) % nbuf)
        scatter_wait((nchunk - 1) % 2, ch_blk - 1, (nblk - 1) % nbuf)
        plsc.subcore_barrier()
        pltpu.sync_copy(acc.at[pl.ds(s * stripe, stripe), :],
                        out_hbm.at[c, pl.ds(s * stripe, stripe), :])

    return k(xd, src64, dst64, w64)


def kernel(X, edge_index, edge_weight, Wu, bu, Wr, br, Wc, bc,
           Wlu, blu, Wlr, blr, Wlc, blc):
    n = X.shape[0]
    e = edge_weight.shape[0]
    src = edge_index[0]
    dst = edge_index[1]

    grain = 256 * _BLK_E
    e_pad = ((e + grain - 1) // grain) * grain
    pad = e_pad - e
    pad_ix = lax.rem(jnp.arange(pad, dtype=src.dtype),
                     jnp.asarray(n, dtype=src.dtype))
    src_p = jnp.concatenate([src, pad_ix])
    dst_p = jnp.concatenate([dst, pad_ix])
    w_p = jnp.pad(edge_weight, (0, pad))
    dst2 = dst_p.reshape(-1, _BLK_E)
    w2 = w_p.reshape(-1, _BLK_E)
    src64 = src_p.reshape(-1, 64)
    dst64 = dst_p.reshape(-1, 64)
    w64 = w_p.reshape(-1, 64)

    sub_grain = _N_SUB * 16
    n_pad = ((n + sub_grain - 1) // sub_grain) * sub_grain

    acc_grain = _N_SUB * 128
    n_acc = ((n + acc_grain - 1) // acc_grain) * acc_grain

    wf, bf = _tc_fold(Wu, Wlu, bu, blu, Wc, Wlc, bc, blc)
    degp = _sc_deg(dst2, w2, n_pad)
    degp3 = degp[:, :n, None]
    xd, dinv = _tc_xd(degp3, X, blk=1000)
    p2 = _sc_spmm(xd, src64, dst64, w64, n_acc)
    return _tc_final(p2[:, :n, :], xd, dinv, wf, bf, blk=1000)
